# baseline (speedup 1.0000x reference)
import functools

import jax
import jax.numpy as jnp
from jax import lax
from jax.experimental import pallas as pl
from jax.experimental.pallas import tpu as pltpu
from jax.experimental.pallas import tpu_sc as plsc

N = 10000
F = 64
NP = 10112
ROWS = NP // 16
NW = 32
BLK = 64
NBLK = 158
EPW = NBLK * BLK
EPAD = NW * EPW

_mesh = plsc.VectorSubcoreMesh(core_axis_name="c", subcore_axis_name="s")
_sc_params = pltpu.CompilerParams(use_tc_tiling_on_sc=False)


@functools.partial(
    pl.kernel,
    out_type=jax.ShapeDtypeStruct((2 * NP,), jnp.float32),
    mesh=_mesh,
    scratch_types=[
        pltpu.VMEM((NBLK, BLK), jnp.int32),
        pltpu.VMEM((BLK,), jnp.float32),
        pltpu.VMEM((ROWS,), jnp.float32),
        pltpu.VMEM_SHARED((NP,), jnp.float32),
    ],
    compiler_params=_sc_params,
)
def _deg_kernel(dst3, zcol, deg_out, idx_v, ones_v, bnc_v, deg_acc):
    c = lax.axis_index("c")
    s = lax.axis_index("s")
    w = c * 16 + s
    pltpu.sync_copy(zcol.at[pl.ds(s * ROWS, ROWS)], bnc_v)
    pltpu.sync_copy(bnc_v, deg_acc.at[pl.ds(s * ROWS, ROWS)])
    pltpu.sync_copy(dst3.at[w], idx_v)
    for i in range(BLK // 16):
        ones_v[pl.ds(i * 16, 16)] = jnp.ones((16,), jnp.float32)
    plsc.subcore_barrier()

    def body(j, carry):
        pltpu.sync_copy(ones_v, deg_acc.at[idx_v.at[j]], add=True)
        return carry

    lax.fori_loop(0, NBLK, body, 0)
    plsc.subcore_barrier()
    pltpu.sync_copy(deg_acc.at[pl.ds(s * ROWS, ROWS)], bnc_v)
    pltpu.sync_copy(bnc_v, deg_out.at[pl.ds(c * NP + s * ROWS, ROWS)])


@functools.partial(
    pl.kernel,
    out_type=(
        jax.ShapeDtypeStruct((2, NP, F), jnp.float32),
        jax.ShapeDtypeStruct((2 * NP,), jnp.float32),
    ),
    mesh=_mesh,
    scratch_types=[
        pltpu.VMEM((NBLK, BLK), jnp.int32),
        pltpu.VMEM((NBLK, BLK), jnp.int32),
        pltpu.VMEM((BLK,), jnp.float32),
        pltpu.VMEM((BLK, F), jnp.float32),
        pltpu.VMEM((ROWS, F), jnp.float32),
        pltpu.VMEM((ROWS,), jnp.float32),
        pltpu.VMEM_SHARED((NP, F), jnp.float32),
        pltpu.VMEM_SHARED((NP,), jnp.float32),
        pltpu.SemaphoreType.DMA,
    ],
    compiler_params=_sc_params,
)
def _edge_kernel(src3, dst3, hs_hbm, dis_hbm, zbig, zcol,
                 tmp_out, s_out,
                 src_v, dst_v, disg_v, gbuf, obuf, cbuf,
                 acc, s_acc, sem):
    c = lax.axis_index("c")
    s = lax.axis_index("s")
    w = c * 16 + s
    pltpu.sync_copy(zbig.at[pl.ds(s * ROWS, ROWS)], obuf)
    pltpu.sync_copy(obuf, acc.at[pl.ds(s * ROWS, ROWS)])
    pltpu.sync_copy(zcol.at[pl.ds(s * ROWS, ROWS)], cbuf)
    pltpu.sync_copy(cbuf, s_acc.at[pl.ds(s * ROWS, ROWS)])
    pltpu.sync_copy(src3.at[w], src_v)
    pltpu.sync_copy(dst3.at[w], dst_v)
    plsc.subcore_barrier()

    def body(j, carry):
        pltpu.async_copy(hs_hbm.at[src_v.at[j]], gbuf, sem).wait()
        pltpu.sync_copy(gbuf, acc.at[dst_v.at[j]], add=True)
        pltpu.async_copy(dis_hbm.at[dst_v.at[j]], disg_v, sem).wait()
        pltpu.sync_copy(disg_v, s_acc.at[src_v.at[j]], add=True)
        return carry

    lax.fori_loop(0, NBLK, body, 0)
    plsc.subcore_barrier()
    pltpu.sync_copy(acc.at[pl.ds(s * ROWS, ROWS)], obuf)
    pltpu.sync_copy(obuf, tmp_out.at[c, pl.ds(s * ROWS, ROWS)])
    pltpu.sync_copy(s_acc.at[pl.ds(s * ROWS, ROWS)], cbuf)
    pltpu.sync_copy(cbuf, s_out.at[pl.ds(c * NP + s * ROWS, ROWS)])


def _ka_body(deg_ref, x2_ref, wg1_ref, hs_ref, dis_ref):
    deg = deg_ref[0] + deg_ref[1] + 1.0
    dis = lax.rsqrt(deg)
    h = jnp.dot(x2_ref[...], wg1_ref[...], preferred_element_type=jnp.float32)
    hs_ref[...] = h * dis
    dis_ref[...] = dis


def _tc_a(deg3, x2, wg1):
    blk = 1000
    return pl.pallas_call(
        _ka_body,
        grid=(N // blk,),
        in_specs=[
            pl.BlockSpec((2, blk, 1), lambda i: (0, i, 0)),
            pl.BlockSpec((blk, 128), lambda i: (i, 0)),
            pl.BlockSpec((128, F), lambda i: (0, 0)),
        ],
        out_specs=[
            pl.BlockSpec((blk, F), lambda i: (i, 0)),
            pl.BlockSpec((blk, 1), lambda i: (i, 0)),
        ],
        out_shape=[
            jax.ShapeDtypeStruct((N, F), jnp.float32),
            jax.ShapeDtypeStruct((N, 1), jnp.float32),
        ],
    )(deg3, x2, wg1)


def _kb_body(tmp_ref, s_ref, hs_ref, dis_ref, bg1_ref, wg2_ref, bg2_ref,
             x1_ref, w1_ref, b1_ref, gamma_ref, beta_ref,
             wf1_ref, bf1_ref, wf2_ref, bf2_ref, out_ref, racc):
    i = pl.program_id(0)
    nblk = pl.num_programs(0)

    @pl.when(i == 0)
    def _():
        racc[...] = jnp.zeros_like(racc)

    t = tmp_ref[0] + tmp_ref[1]
    dis = dis_ref[...]
    h1 = jnp.maximum(dis * (t + hs_ref[...]) + bg1_ref[...], 0.0)
    sv = s_ref[0] + s_ref[1]
    wcol = dis * (sv + dis)
    contrib = (wcol * h1).reshape(-1, 8, F).sum(axis=0)
    racc[...] += contrib

    @pl.when(i == nblk - 1)
    def _():
        r = racc[...].sum(axis=0, keepdims=True)
        g = jnp.dot(r / N, wg2_ref[...],
                    preferred_element_type=jnp.float32) + bg2_ref[...]
        z = jnp.dot(x1_ref[...], w1_ref[...],
                    preferred_element_type=jnp.float32) + b1_ref[...]
        mu = jnp.mean(z, axis=0, keepdims=True)
        var = jnp.mean((z - mu) * (z - mu), axis=0, keepdims=True)
        dnn = jnp.maximum((z - mu) * lax.rsqrt(var + 1e-5) * gamma_ref[...]
                          + beta_ref[...], 0.0)
        v1 = jnp.dot(wf1_ref[0:F, :], wf2_ref[...],
                     preferred_element_type=jnp.float32)
        v2 = jnp.dot(wf1_ref[F:2 * F, :], wf2_ref[...],
                     preferred_element_type=jnp.float32)
        gterm = jnp.dot(g, v2, preferred_element_type=jnp.float32)
        cterm = jnp.dot(bf1_ref[...], wf2_ref[...],
                        preferred_element_type=jnp.float32) + bf2_ref[...]
        out_ref[...] = jnp.dot(dnn, v1,
                               preferred_element_type=jnp.float32) \
            + gterm + cterm


def _tc_b(tmp_part, s3, hs, dis, bg1, wg2, bg2, x1, w1, b1, gamma, beta,
          wf1, bf1, wf2, bf2):
    blk = 1000
    c0 = lambda i: (0, 0)
    return pl.pallas_call(
        _kb_body,
        grid=(N // blk,),
        in_specs=[
            pl.BlockSpec((2, blk, F), lambda i: (0, i, 0)),
            pl.BlockSpec((2, blk, 1), lambda i: (0, i, 0)),
            pl.BlockSpec((blk, F), lambda i: (i, 0)),
            pl.BlockSpec((blk, 1), lambda i: (i, 0)),
            pl.BlockSpec((1, F), c0),
            pl.BlockSpec((F, F), c0),
            pl.BlockSpec((1, F), c0),
            pl.BlockSpec((256, 512), c0),
            pl.BlockSpec((512, F), c0),
            pl.BlockSpec((1, F), c0),
            pl.BlockSpec((1, F), c0),
            pl.BlockSpec((1, F), c0),
            pl.BlockSpec((2 * F, F), c0),
            pl.BlockSpec((1, F), c0),
            pl.BlockSpec((F, 1), c0),
            pl.BlockSpec((1, 1), c0),
        ],
        out_specs=pl.BlockSpec((256, 1), c0),
        out_shape=jax.ShapeDtypeStruct((256, 1), jnp.float32),
        scratch_shapes=[pltpu.VMEM((8, F), jnp.float32)],
    )(tmp_part, s3, hs, dis, bg1, wg2, bg2, x1, w1, b1, gamma, beta,
      wf1, bf1, wf2, bf2)


def kernel(x1, x2, edge_index, W1, b1, gamma, beta, Wg1, bg1, Wg2, bg2,
           Wf1, bf1, Wf2, bf2):
    E = edge_index.shape[1]
    src = edge_index[0].astype(jnp.int32)
    dst = edge_index[1].astype(jnp.int32)
    pad = EPAD - E
    src3 = jnp.concatenate([src, jnp.zeros((pad,), jnp.int32)]).reshape(
        NW, NBLK, BLK)
    dst3 = jnp.concatenate([dst, jnp.full((pad,), N, jnp.int32)]).reshape(
        NW, NBLK, BLK)
    zcol = jnp.zeros((NP,), jnp.float32)
    zbig = jnp.zeros((NP, F), jnp.float32)

    deg_part = _deg_kernel(dst3, zcol).reshape(2, NP)
    hs, dis = _tc_a(deg_part.reshape(2, NP, 1), x2, Wg1)
    dis_pad = jnp.concatenate([dis[:, 0], jnp.zeros((NP - N,), jnp.float32)])
    tmp_part, s_part = _edge_kernel(src3, dst3, hs, dis_pad, zbig, zcol)
    s_part = s_part.reshape(2, NP)

    out = _tc_b(tmp_part, s_part.reshape(2, NP, 1), hs, dis,
                bg1.reshape(1, F), Wg2, bg2.reshape(1, F),
                x1, W1, b1.reshape(1, F), gamma.reshape(1, F),
                beta.reshape(1, F), Wf1, bf1.reshape(1, F), Wf2,
                bf2.reshape(1, 1))
    return out[:, 0]


# speedup vs baseline: 1.7486x; 1.7486x over previous
import functools

import jax
import jax.numpy as jnp
from jax import lax
from jax.experimental import pallas as pl
from jax.experimental.pallas import tpu as pltpu
from jax.experimental.pallas import tpu_sc as plsc

N = 10000
F = 64
NP = 10112
ROWS = NP // 16
NW = 32
BLK = 128
NBLK = 79
EPW = NBLK * BLK
EPAD = NW * EPW

_mesh = plsc.VectorSubcoreMesh(core_axis_name="c", subcore_axis_name="s")
_sc_params = pltpu.CompilerParams(use_tc_tiling_on_sc=False)


@functools.partial(
    pl.kernel,
    out_type=jax.ShapeDtypeStruct((2 * NP,), jnp.float32),
    mesh=_mesh,
    scratch_types=[
        pltpu.VMEM((NBLK, BLK), jnp.int32),
        pltpu.VMEM((BLK,), jnp.float32),
        pltpu.VMEM((ROWS,), jnp.float32),
        pltpu.VMEM_SHARED((NP,), jnp.float32),
    ],
    compiler_params=_sc_params,
)
def _deg_kernel(dst3, zcol, deg_out, idx_v, ones_v, bnc_v, deg_acc):
    c = lax.axis_index("c")
    s = lax.axis_index("s")
    w = c * 16 + s
    pltpu.sync_copy(zcol.at[pl.ds(s * ROWS, ROWS)], bnc_v)
    pltpu.sync_copy(bnc_v, deg_acc.at[pl.ds(s * ROWS, ROWS)])
    pltpu.sync_copy(dst3.at[w], idx_v)
    for i in range(BLK // 16):
        ones_v[pl.ds(i * 16, 16)] = jnp.ones((16,), jnp.float32)
    plsc.subcore_barrier()

    def body(j, carry):
        pltpu.sync_copy(ones_v, deg_acc.at[idx_v.at[j]], add=True)
        return carry

    lax.fori_loop(0, NBLK, body, 0)
    plsc.subcore_barrier()
    pltpu.sync_copy(deg_acc.at[pl.ds(s * ROWS, ROWS)], bnc_v)
    pltpu.sync_copy(bnc_v, deg_out.at[pl.ds(c * NP + s * ROWS, ROWS)])


@functools.partial(
    pl.kernel,
    out_type=(
        jax.ShapeDtypeStruct((2, NP, F), jnp.float32),
        jax.ShapeDtypeStruct((2 * NP,), jnp.float32),
    ),
    mesh=_mesh,
    scratch_types=[
        pltpu.VMEM((NBLK, BLK), jnp.int32),
        pltpu.VMEM((NBLK, BLK), jnp.int32),
        pltpu.VMEM((2, BLK), jnp.float32),
        pltpu.VMEM((2, BLK, F), jnp.float32),
        pltpu.VMEM((ROWS, F), jnp.float32),
        pltpu.VMEM((ROWS,), jnp.float32),
        pltpu.VMEM_SHARED((NP, F), jnp.float32),
        pltpu.VMEM_SHARED((NP,), jnp.float32),
        pltpu.SemaphoreType.DMA,
        pltpu.SemaphoreType.DMA,
    ],
    compiler_params=_sc_params,
)
def _edge_kernel(src3, dst3, hs_hbm, dis_hbm, zbig, zcol,
                 tmp_out, s_out,
                 src_v, dst_v, disg_v, gbuf, obuf, cbuf,
                 acc, s_acc, sem, dsem):
    c = lax.axis_index("c")
    s = lax.axis_index("s")
    w = c * 16 + s
    pltpu.sync_copy(zbig.at[pl.ds(s * ROWS, ROWS)], obuf)
    pltpu.sync_copy(obuf, acc.at[pl.ds(s * ROWS, ROWS)])
    pltpu.sync_copy(zcol.at[pl.ds(s * ROWS, ROWS)], cbuf)
    pltpu.sync_copy(cbuf, s_acc.at[pl.ds(s * ROWS, ROWS)])
    pltpu.sync_copy(src3.at[w], src_v)
    pltpu.sync_copy(dst3.at[w], dst_v)
    plsc.subcore_barrier()

    gd = pltpu.async_copy(hs_hbm.at[src_v.at[0]], gbuf.at[0], sem)
    dd = pltpu.async_copy(dis_hbm.at[dst_v.at[0]], disg_v.at[0], dsem)
    for j in range(NBLK):
        p = j % 2
        if j + 1 < NBLK:
            gn = pltpu.async_copy(hs_hbm.at[src_v.at[j + 1]],
                                  gbuf.at[1 - p], sem)
            dn = pltpu.async_copy(dis_hbm.at[dst_v.at[j + 1]],
                                  disg_v.at[1 - p], dsem)
        gd.wait()
        pltpu.sync_copy(gbuf.at[p], acc.at[dst_v.at[j]], add=True)
        dd.wait()
        pltpu.sync_copy(disg_v.at[p], s_acc.at[src_v.at[j]], add=True)
        if j + 1 < NBLK:
            gd, dd = gn, dn
    plsc.subcore_barrier()
    pltpu.sync_copy(acc.at[pl.ds(s * ROWS, ROWS)], obuf)
    pltpu.sync_copy(obuf, tmp_out.at[c, pl.ds(s * ROWS, ROWS)])
    pltpu.sync_copy(s_acc.at[pl.ds(s * ROWS, ROWS)], cbuf)
    pltpu.sync_copy(cbuf, s_out.at[pl.ds(c * NP + s * ROWS, ROWS)])


def _ka_body(deg_ref, x2_ref, wg1_ref, hs_ref, dis_ref):
    deg = deg_ref[0] + deg_ref[1] + 1.0
    dis = lax.rsqrt(deg)
    h = jnp.dot(x2_ref[...], wg1_ref[...], preferred_element_type=jnp.float32)
    hs_ref[...] = h * dis
    dis_ref[...] = dis


def _tc_a(deg3, x2, wg1):
    blk = 1000
    return pl.pallas_call(
        _ka_body,
        grid=(N // blk,),
        in_specs=[
            pl.BlockSpec((2, blk, 1), lambda i: (0, i, 0)),
            pl.BlockSpec((blk, 128), lambda i: (i, 0)),
            pl.BlockSpec((128, F), lambda i: (0, 0)),
        ],
        out_specs=[
            pl.BlockSpec((blk, F), lambda i: (i, 0)),
            pl.BlockSpec((blk, 1), lambda i: (i, 0)),
        ],
        out_shape=[
            jax.ShapeDtypeStruct((N, F), jnp.float32),
            jax.ShapeDtypeStruct((N, 1), jnp.float32),
        ],
    )(deg3, x2, wg1)


def _kb_body(tmp_ref, s_ref, hs_ref, dis_ref, bg1_ref, wg2_ref, bg2_ref,
             x1_ref, w1_ref, b1_ref, gamma_ref, beta_ref,
             wf1_ref, bf1_ref, wf2_ref, bf2_ref, out_ref, racc):
    i = pl.program_id(0)
    nblk = pl.num_programs(0)

    @pl.when(i == 0)
    def _():
        racc[...] = jnp.zeros_like(racc)

    t = tmp_ref[0] + tmp_ref[1]
    dis = dis_ref[...]
    h1 = jnp.maximum(dis * (t + hs_ref[...]) + bg1_ref[...], 0.0)
    sv = s_ref[0] + s_ref[1]
    wcol = dis * (sv + dis)
    contrib = (wcol * h1).reshape(-1, 8, F).sum(axis=0)
    racc[...] += contrib

    @pl.when(i == nblk - 1)
    def _():
        r = racc[...].sum(axis=0, keepdims=True)
        g = jnp.dot(r / N, wg2_ref[...],
                    preferred_element_type=jnp.float32) + bg2_ref[...]
        z = jnp.dot(x1_ref[...], w1_ref[...],
                    preferred_element_type=jnp.float32) + b1_ref[...]
        mu = jnp.mean(z, axis=0, keepdims=True)
        var = jnp.mean((z - mu) * (z - mu), axis=0, keepdims=True)
        dnn = jnp.maximum((z - mu) * lax.rsqrt(var + 1e-5) * gamma_ref[...]
                          + beta_ref[...], 0.0)
        v1 = jnp.dot(wf1_ref[0:F, :], wf2_ref[...],
                     preferred_element_type=jnp.float32)
        v2 = jnp.dot(wf1_ref[F:2 * F, :], wf2_ref[...],
                     preferred_element_type=jnp.float32)
        gterm = jnp.dot(g, v2, preferred_element_type=jnp.float32)
        cterm = jnp.dot(bf1_ref[...], wf2_ref[...],
                        preferred_element_type=jnp.float32) + bf2_ref[...]
        out_ref[...] = jnp.dot(dnn, v1,
                               preferred_element_type=jnp.float32) \
            + gterm + cterm


def _tc_b(tmp_part, s3, hs, dis, bg1, wg2, bg2, x1, w1, b1, gamma, beta,
          wf1, bf1, wf2, bf2):
    blk = 1000
    c0 = lambda i: (0, 0)
    return pl.pallas_call(
        _kb_body,
        grid=(N // blk,),
        in_specs=[
            pl.BlockSpec((2, blk, F), lambda i: (0, i, 0)),
            pl.BlockSpec((2, blk, 1), lambda i: (0, i, 0)),
            pl.BlockSpec((blk, F), lambda i: (i, 0)),
            pl.BlockSpec((blk, 1), lambda i: (i, 0)),
            pl.BlockSpec((1, F), c0),
            pl.BlockSpec((F, F), c0),
            pl.BlockSpec((1, F), c0),
            pl.BlockSpec((256, 512), c0),
            pl.BlockSpec((512, F), c0),
            pl.BlockSpec((1, F), c0),
            pl.BlockSpec((1, F), c0),
            pl.BlockSpec((1, F), c0),
            pl.BlockSpec((2 * F, F), c0),
            pl.BlockSpec((1, F), c0),
            pl.BlockSpec((F, 1), c0),
            pl.BlockSpec((1, 1), c0),
        ],
        out_specs=pl.BlockSpec((256, 1), c0),
        out_shape=jax.ShapeDtypeStruct((256, 1), jnp.float32),
        scratch_shapes=[pltpu.VMEM((8, F), jnp.float32)],
    )(tmp_part, s3, hs, dis, bg1, wg2, bg2, x1, w1, b1, gamma, beta,
      wf1, bf1, wf2, bf2)


def kernel(x1, x2, edge_index, W1, b1, gamma, beta, Wg1, bg1, Wg2, bg2,
           Wf1, bf1, Wf2, bf2):
    E = edge_index.shape[1]
    src = edge_index[0].astype(jnp.int32)
    dst = edge_index[1].astype(jnp.int32)
    pad = EPAD - E
    src3 = jnp.concatenate([src, jnp.zeros((pad,), jnp.int32)]).reshape(
        NW, NBLK, BLK)
    dst3 = jnp.concatenate([dst, jnp.full((pad,), N, jnp.int32)]).reshape(
        NW, NBLK, BLK)
    zcol = jnp.zeros((NP,), jnp.float32)
    zbig = jnp.zeros((NP, F), jnp.float32)

    deg_part = _deg_kernel(dst3, zcol).reshape(2, NP)
    hs, dis = _tc_a(deg_part.reshape(2, NP, 1), x2, Wg1)
    dis_pad = jnp.concatenate([dis[:, 0], jnp.zeros((NP - N,), jnp.float32)])
    tmp_part, s_part = _edge_kernel(src3, dst3, hs, dis_pad, zbig, zcol)
    s_part = s_part.reshape(2, NP)

    out = _tc_b(tmp_part, s_part.reshape(2, NP, 1), hs, dis,
                bg1.reshape(1, F), Wg2, bg2.reshape(1, F),
                x1, W1, b1.reshape(1, F), gamma.reshape(1, F),
                beta.reshape(1, F), Wf1, bf1.reshape(1, F), Wf2,
                bf2.reshape(1, 1))
    return out[:, 0]
